# baseline (speedup 1.0000x reference)
import jax
import jax.numpy as jnp
from jax.experimental import pallas as pl

_FC, _FH, _FW = 8, 4, 4
_AH, _AW = 8, 8


def _split3(a):
    hi = a.astype(jnp.bfloat16).astype(jnp.float32)
    lo = (a - hi).astype(jnp.bfloat16).astype(jnp.float32)
    return hi, lo


def _dot3(a, b, dims):
    a_hi, a_lo = _split3(a)
    b_hi, b_lo = _split3(b)
    dn = (dims, ((), ()))
    kw = dict(preferred_element_type=jnp.float32,
              precision=jax.lax.Precision.DEFAULT)
    return (jax.lax.dot_general(a_hi, b_hi, dn, **kw)
            + jax.lax.dot_general(a_lo, b_hi, dn, **kw)
            + jax.lax.dot_general(a_hi, b_lo, dn, **kw))


def kernel(x, Wp, bp, Wm, bm, alpha, beta):
    f32 = jnp.float32
    n, c, h, w = x.shape
    fc, fh, fw = _FC, _FH, _FW
    ah, aw = _AH, _AW
    sh, sw = h // fh, w // fw
    L = sh * sw
    o2 = Wp.shape[0]
    scn = o2 // fc
    half = scn // 2
    S = ah * aw
    kh, kw = sh // ah, sw // aw
    co = Wm.shape[0]
    T = n * fh * fw

    bp2 = bp.reshape(o2, 1).astype(f32)
    bm2 = bm.reshape(co, 1).astype(f32)
    ab = jnp.concatenate([alpha.reshape(-1), beta.reshape(-1)]).reshape(1, 2).astype(f32)

    DEF = jax.lax.Precision.DEFAULT
    HIGHEST = jax.lax.Precision.HIGHEST

    def body(x_ref, wp_ref, bp_ref, wm_ref, bm_ref, ab_ref, o_ref):
        xcm = x_ref[0]
        y = jax.lax.dot_general(
            wp_ref[...], xcm, (((1,), (0,)), ((), ())),
            preferred_element_type=f32, precision=DEF) + bp_ref[...]

        pix = jax.lax.broadcasted_iota(jnp.int32, (L, S), 0)
        sid = jax.lax.broadcasted_iota(jnp.int32, (L, S), 1)
        win = (pix // (sw * kh)) * aw + (pix % sw) // kw
        P = jnp.where(win == sid, f32(1.0 / (kh * kw)), f32(0.0))
        anchor_all = jax.lax.dot_general(
            y, P, (((1,), (0,)), ((), ())),
            preferred_element_type=f32, precision=HIGHEST)

        al = ab_ref[0, 0]
        be = ab_ref[0, 1]
        iota_s = jax.lax.broadcasted_iota(jnp.int32, (S, L), 0)
        ones_l = jnp.ones((1, L), f32)
        ones_s = jnp.ones((1, S), f32)

        disp_parts = []
        for g in range(fc):
            yg = y[scn * g:scn * (g + 1), :]
            anchor = anchor_all[scn * g:scn * (g + 1), :]
            xp, xv = yg[:half, :], yg[half:, :]
            apt, av = anchor[:half, :], anchor[half:, :]
            xn = xp / jnp.maximum(
                jnp.sqrt(jnp.sum(xp * xp, axis=0, keepdims=True)), 1e-12)
            an = apt / jnp.maximum(
                jnp.sqrt(jnp.sum(apt * apt, axis=0, keepdims=True)), 1e-12)
            sim = jax.lax.dot_general(
                an, xn, (((0,), (0,)), ((), ())),
                preferred_element_type=f32, precision=DEF)
            z = jax.nn.sigmoid(al * sim + be)
            maxv = jnp.max(z, axis=0, keepdims=True)
            idx = jnp.min(jnp.where(z == maxv, iota_s, S),
                          axis=0, keepdims=True)
            W1 = jnp.where(iota_s == idx, maxv, f32(0.0))
            catx = jnp.concatenate([xv, ones_l], axis=0)
            cata = jnp.concatenate([av, ones_s], axis=0)
            agg = cata + _dot3(catx, W1, ((1,), (1,)))
            aggn = agg[:half, :] / agg[half:half + 1, :]
            disp = _dot3(aggn, W1, ((1,), (0,)))
            disp_parts.append(disp)

        dispf = jnp.concatenate(disp_parts, axis=0)
        out = jax.lax.dot_general(
            wm_ref[...], dispf, (((1,), (0,)), ((), ())),
            preferred_element_type=f32, precision=DEF) + bm_ref[...]
        o_ref[0] = out

    call = pl.pallas_call(
        body,
        grid=(n * fw,),
        in_specs=[
            pl.BlockSpec((1, c, L), lambda i: (i, 0, 0)),
            pl.BlockSpec((o2, c), lambda i: (0, 0)),
            pl.BlockSpec((o2, 1), lambda i: (0, 0)),
            pl.BlockSpec((co, co), lambda i: (0, 0)),
            pl.BlockSpec((co, 1), lambda i: (0, 0)),
            pl.BlockSpec((1, 2), lambda i: (0, 0)),
        ],
        out_specs=pl.BlockSpec((1, co, L), lambda i: (i, 0, 0)),
        out_shape=jax.ShapeDtypeStruct((n * fw, co, L), f32),
    )

    out_bands = []
    for i in range(fh):
        xi = x[:, :, sh * i:sh * (i + 1), :]
        xti = xi.reshape(n, c, sh, fw, sw).transpose(0, 3, 1, 2, 4)
        xti = xti.reshape(n * fw, c, L)
        ot = call(xti, Wp, bp2, Wm, bm2, ab)
        ot = ot.reshape(n, fw, co, sh, sw).transpose(0, 2, 3, 1, 4)
        out_bands.append(ot.reshape(n, co, sh, w))
    return jnp.concatenate(out_bands, axis=2)


# speedup vs baseline: 1.1127x; 1.1127x over previous
import jax
import jax.numpy as jnp
from jax.experimental import pallas as pl
from jax.experimental.pallas import tpu as pltpu

_FC, _FH, _FW = 8, 4, 4
_AH, _AW = 8, 8


def _split3(a):
    hi = a.astype(jnp.bfloat16).astype(jnp.float32)
    lo = (a - hi).astype(jnp.bfloat16).astype(jnp.float32)
    return hi, lo


def _dot3(a, b, dims):
    a_hi, a_lo = _split3(a)
    b_hi, b_lo = _split3(b)
    dn = (dims, ((), ()))
    kw = dict(preferred_element_type=jnp.float32,
              precision=jax.lax.Precision.DEFAULT)
    return (jax.lax.dot_general(a_hi, b_hi, dn, **kw)
            + jax.lax.dot_general(a_lo, b_hi, dn, **kw)
            + jax.lax.dot_general(a_hi, b_lo, dn, **kw))


def kernel(x, Wp, bp, Wm, bm, alpha, beta):
    f32 = jnp.float32
    n, c, h, w = x.shape
    fc, fh, fw = _FC, _FH, _FW
    ah, aw = _AH, _AW
    sh, sw = h // fh, w // fw
    L = sh * sw
    o2 = Wp.shape[0]
    scn = o2 // fc
    half = scn // 2
    S = ah * aw
    kh, kw = sh // ah, sw // aw
    co = Wm.shape[0]
    T = n * fh * fw

    xt = x.reshape(n, c, fh, sh, fw, sw).transpose(0, 2, 4, 1, 3, 5).reshape(T, c, L)
    bp2 = bp.reshape(o2, 1).astype(f32)
    bm2 = bm.reshape(co, 1).astype(f32)
    ab = jnp.concatenate([alpha.reshape(-1), beta.reshape(-1)]).reshape(1, 2).astype(f32)

    DEF = jax.lax.Precision.DEFAULT
    HIGHEST = jax.lax.Precision.HIGHEST

    def body(x_ref, wp_ref, bp_ref, wm_ref, bm_ref, ab_ref, o_ref):
        xcm = x_ref[0]
        y = jax.lax.dot_general(
            wp_ref[...], xcm, (((1,), (0,)), ((), ())),
            preferred_element_type=f32, precision=DEF) + bp_ref[...]

        pix = jax.lax.broadcasted_iota(jnp.int32, (L, S), 0)
        sid = jax.lax.broadcasted_iota(jnp.int32, (L, S), 1)
        win = (pix // (sw * kh)) * aw + (pix % sw) // kw
        P = jnp.where(win == sid, f32(1.0 / (kh * kw)), f32(0.0))
        anchor_all = jax.lax.dot_general(
            y, P, (((1,), (0,)), ((), ())),
            preferred_element_type=f32, precision=HIGHEST)

        al = ab_ref[0, 0]
        be = ab_ref[0, 1]
        iota_s = jax.lax.broadcasted_iota(jnp.int32, (S, L), 0)
        ones_l = jnp.ones((1, L), f32)
        ones_s = jnp.ones((1, S), f32)

        disp_parts = []
        for g in range(fc):
            yg = y[scn * g:scn * (g + 1), :]
            anchor = anchor_all[scn * g:scn * (g + 1), :]
            xp, xv = yg[:half, :], yg[half:, :]
            apt, av = anchor[:half, :], anchor[half:, :]
            xn = xp / jnp.maximum(
                jnp.sqrt(jnp.sum(xp * xp, axis=0, keepdims=True)), 1e-12)
            an = apt / jnp.maximum(
                jnp.sqrt(jnp.sum(apt * apt, axis=0, keepdims=True)), 1e-12)
            sim = jax.lax.dot_general(
                an, xn, (((0,), (0,)), ((), ())),
                preferred_element_type=f32, precision=DEF)
            z = jax.nn.sigmoid(al * sim + be)
            maxv = jnp.max(z, axis=0, keepdims=True)
            idx = jnp.min(jnp.where(z == maxv, iota_s, S),
                          axis=0, keepdims=True)
            W1 = jnp.where(iota_s == idx, maxv, f32(0.0))
            catx = jnp.concatenate([xv, ones_l], axis=0)
            cata = jnp.concatenate([av, ones_s], axis=0)
            agg = cata + _dot3(catx, W1, ((1,), (1,)))
            aggn = agg[:half, :] / agg[half:half + 1, :]
            disp = _dot3(aggn, W1, ((1,), (0,)))
            disp_parts.append(disp)

        dispf = jnp.concatenate(disp_parts, axis=0)
        out = jax.lax.dot_general(
            wm_ref[...], dispf, (((1,), (0,)), ((), ())),
            preferred_element_type=f32, precision=DEF) + bm_ref[...]
        o_ref[0] = out

    out_t = pl.pallas_call(
        body,
        grid=(T,),
        in_specs=[
            pl.BlockSpec((1, c, L), lambda i: (i, 0, 0)),
            pl.BlockSpec((o2, c), lambda i: (0, 0)),
            pl.BlockSpec((o2, 1), lambda i: (0, 0)),
            pl.BlockSpec((co, co), lambda i: (0, 0)),
            pl.BlockSpec((co, 1), lambda i: (0, 0)),
            pl.BlockSpec((1, 2), lambda i: (0, 0)),
        ],
        out_specs=pl.BlockSpec((1, co, L), lambda i: (i, 0, 0)),
        out_shape=jax.ShapeDtypeStruct((T, co, L), f32),
        compiler_params=pltpu.CompilerParams(
            dimension_semantics=("parallel",)),
    )(xt, Wp, bp2, Wm, bm2, ab)

    out = out_t.reshape(n, fh, fw, co, sh, sw).transpose(0, 3, 1, 4, 2, 5)
    return out.reshape(n, co, h, w)


# speedup vs baseline: 1.1264x; 1.0123x over previous
import jax
import jax.numpy as jnp
from jax.experimental import pallas as pl
from jax.experimental.pallas import tpu as pltpu

_FC, _FH, _FW = 8, 4, 4
_AH, _AW = 8, 8


def _split3(a):
    hi = a.astype(jnp.bfloat16).astype(jnp.float32)
    lo = (a - hi).astype(jnp.bfloat16).astype(jnp.float32)
    return hi, lo


def _dot3(a, b, dims):
    a_hi, a_lo = _split3(a)
    b_hi, b_lo = _split3(b)
    dn = (dims, ((), ()))
    kw = dict(preferred_element_type=jnp.float32,
              precision=jax.lax.Precision.DEFAULT)
    return (jax.lax.dot_general(a_hi, b_hi, dn, **kw)
            + jax.lax.dot_general(a_lo, b_hi, dn, **kw)
            + jax.lax.dot_general(a_hi, b_lo, dn, **kw))


def kernel(x, Wp, bp, Wm, bm, alpha, beta):
    f32 = jnp.float32
    n, c, h, w = x.shape
    fc, fh, fw = _FC, _FH, _FW
    ah, aw = _AH, _AW
    sh, sw = h // fh, w // fw
    L = sh * sw
    o2 = Wp.shape[0]
    scn = o2 // fc
    half = scn // 2
    S = ah * aw
    kh, kw = sh // ah, sw // aw
    co = Wm.shape[0]
    T = n * fh * fw

    xt = x.reshape(n, c, fh, sh, fw, sw).transpose(0, 2, 4, 1, 3, 5).reshape(T, c, L)
    bp2 = bp.reshape(o2, 1).astype(f32)
    bm2 = bm.reshape(co, 1).astype(f32)
    ab = jnp.concatenate([alpha.reshape(-1), beta.reshape(-1)]).reshape(1, 2).astype(f32)

    DEF = jax.lax.Precision.DEFAULT
    HIGHEST = jax.lax.Precision.HIGHEST

    def body(x_ref, wp_ref, bp_ref, wm_ref, bm_ref, ab_ref, o_ref):
        xcm = x_ref[0]
        y = jax.lax.dot_general(
            wp_ref[...], xcm, (((1,), (0,)), ((), ())),
            preferred_element_type=f32, precision=DEF) + bp_ref[...]

        pix = jax.lax.broadcasted_iota(jnp.int32, (L, S), 0)
        sid = jax.lax.broadcasted_iota(jnp.int32, (L, S), 1)
        win = (pix // (sw * kh)) * aw + (pix % sw) // kw
        P = jnp.where(win == sid, f32(1.0 / (kh * kw)), f32(0.0))
        anchor_all = jax.lax.dot_general(
            y, P, (((1,), (0,)), ((), ())),
            preferred_element_type=f32, precision=HIGHEST)

        al = ab_ref[0, 0]
        be = ab_ref[0, 1]
        iota_s = jax.lax.broadcasted_iota(jnp.int32, (S, L), 0)
        ones_l = jnp.ones((1, L), f32)
        ones_s = jnp.ones((1, S), f32)

        disp_parts = []
        for g in range(fc):
            yg = y[scn * g:scn * (g + 1), :]
            anchor = anchor_all[scn * g:scn * (g + 1), :]
            xp, xv = yg[:half, :], yg[half:, :]
            apt, av = anchor[:half, :], anchor[half:, :]
            xn = xp / jnp.maximum(
                jnp.sqrt(jnp.sum(xp * xp, axis=0, keepdims=True)), 1e-12)
            an = apt / jnp.maximum(
                jnp.sqrt(jnp.sum(apt * apt, axis=0, keepdims=True)), 1e-12)
            sim = jax.lax.dot_general(
                an, xn, (((0,), (0,)), ((), ())),
                preferred_element_type=f32, precision=DEF)
            z2 = al * sim + be
            m2 = jnp.max(z2, axis=0, keepdims=True)
            idx = jnp.min(jnp.where(z2 == m2, iota_s, S),
                          axis=0, keepdims=True)
            maxv = jax.nn.sigmoid(m2)
            W1 = jnp.where(iota_s == idx, maxv, f32(0.0))
            catx = jnp.concatenate([xv, ones_l], axis=0)
            cata = jnp.concatenate([av, ones_s], axis=0)
            agg = cata + _dot3(catx, W1, ((1,), (1,)))
            aggn = agg[:half, :] / agg[half:half + 1, :]
            disp = _dot3(aggn, W1, ((1,), (0,)))
            disp_parts.append(disp)

        dispf = jnp.concatenate(disp_parts, axis=0)
        out = jax.lax.dot_general(
            wm_ref[...], dispf, (((1,), (0,)), ((), ())),
            preferred_element_type=f32, precision=DEF) + bm_ref[...]
        o_ref[0] = out

    out_t = pl.pallas_call(
        body,
        grid=(T,),
        in_specs=[
            pl.BlockSpec((1, c, L), lambda i: (i, 0, 0)),
            pl.BlockSpec((o2, c), lambda i: (0, 0)),
            pl.BlockSpec((o2, 1), lambda i: (0, 0)),
            pl.BlockSpec((co, co), lambda i: (0, 0)),
            pl.BlockSpec((co, 1), lambda i: (0, 0)),
            pl.BlockSpec((1, 2), lambda i: (0, 0)),
        ],
        out_specs=pl.BlockSpec((1, co, L), lambda i: (i, 0, 0)),
        out_shape=jax.ShapeDtypeStruct((T, co, L), f32),
        compiler_params=pltpu.CompilerParams(
            dimension_semantics=("parallel",)),
    )(xt, Wp, bp2, Wm, bm2, ab)

    out = out_t.reshape(n, fh, fw, co, sh, sw).transpose(0, 3, 1, 4, 2, 5)
    return out.reshape(n, co, h, w)
